# baseline (speedup 1.0000x reference)
import functools

import jax
import jax.numpy as jnp
from jax import lax
from jax.experimental import pallas as pl
from jax.experimental.pallas import tpu as pltpu
from jax.experimental.pallas import tpu_sc as plsc

_N = 10000
_E = 160000
_NW = 32
_NT = 320
_NPAD = _NW * _NT
_C = 8000
_NCH = _E // _C
_CP = _C + 64
_TRASH = _NT
_ACC_ROWS = _NT + 8
_R = 1000


def _lrelu(v):
    return jnp.where(v > 0, v, 0.2 * v)


def _partition(src, dst):
    mesh = plsc.VectorSubcoreMesh(core_axis_name="c", subcore_axis_name="s")

    @functools.partial(
        pl.kernel,
        mesh=mesh,
        out_type=(
            jax.ShapeDtypeStruct((_NW, _NCH, _CP), jnp.int32),
            jax.ShapeDtypeStruct((_NW, _NCH, _CP), jnp.int32),
            jax.ShapeDtypeStruct((_NW, _NCH, 16), jnp.int32),
        ),
        scratch_types=[
            pltpu.VMEM((_C,), jnp.int32),
            pltpu.VMEM((_C,), jnp.int32),
            pltpu.VMEM((_CP,), jnp.int32),
            pltpu.VMEM((_CP,), jnp.int32),
            pltpu.VMEM((16,), jnp.int32),
        ],
        compiler_params=pltpu.CompilerParams(
            needs_layout_passes=False, use_tc_tiling_on_sc=False),
    )
    def part_kernel(src_hbm, dst_hbm, osrc, odst, ocnt, sv, dv, ms, md, cv):
        wid = lax.axis_index("s") * mesh.num_cores + lax.axis_index("c")
        lo = wid * _NT
        hi = lo + _NT
        lane = lax.iota(jnp.int32, 16)

        def ms_init(i, _):
            ms[pl.ds(i * 16, 16)] = jnp.zeros((16,), jnp.int32)
            return 0

        lax.fori_loop(0, _CP // 16, ms_init, 0)

        def chunk_body(ch, _):
            pltpu.sync_copy(src_hbm.at[pl.ds(ch * _C, _C)], sv)
            pltpu.sync_copy(dst_hbm.at[pl.ds(ch * _C, _C)], dv)

            def grp_body(g, cnt):
                d = dv[pl.ds(g * 16, 16)]
                s = sv[pl.ds(g * 16, 16)]
                m = (d >= lo) & (d < hi)
                mi = m.astype(jnp.int32)
                pos = plsc.cumsum(mi) - mi + cnt
                plsc.store_scatter(ms, [pos], s, mask=m)
                plsc.store_scatter(md, [pos], d - lo, mask=m)
                return cnt + jnp.sum(mi)

            cnt = lax.fori_loop(0, _C // 16, grp_body, jnp.int32(0))
            for tg in range(4):
                tpos = cnt + tg * 16 + lane
                plsc.store_scatter(ms, [tpos], jnp.zeros((16,), jnp.int32))
                plsc.store_scatter(md, [tpos], jnp.full((16,), _TRASH, jnp.int32))
            cv[pl.ds(0, 16)] = jnp.full((16,), 1, jnp.int32) * cnt
            pltpu.sync_copy(ms, osrc.at[wid, ch])
            pltpu.sync_copy(md, odst.at[wid, ch])
            pltpu.sync_copy(cv, ocnt.at[wid, ch])
            return 0

        lax.fori_loop(0, _NCH, chunk_body, 0)

    return part_kernel(src, dst)


def _segmax(feat, srcl, dstl, cnts, F):
    mesh = plsc.VectorSubcoreMesh(core_axis_name="c", subcore_axis_name="s")
    nfc = F // 16

    @functools.partial(
        pl.kernel,
        mesh=mesh,
        out_type=jax.ShapeDtypeStruct((_NPAD * F,), jnp.float32),
        scratch_types=[
            pltpu.VMEM((_ACC_ROWS * F,), jnp.float32),
            pltpu.VMEM((_CP,), jnp.int32),
            pltpu.VMEM((_CP,), jnp.int32),
            pltpu.VMEM((16,), jnp.int32),
            pltpu.VMEM((32, F), jnp.float32),
            pltpu.SemaphoreType.DMA,
        ],
        compiler_params=pltpu.CompilerParams(
            needs_layout_passes=False, use_tc_tiling_on_sc=False),
    )
    def seg_kernel(feat_hbm, srcl_hbm, dstl_hbm, cnt_hbm, out_hbm,
                   acc, ms, md, cv, rows, sem):
        wid = lax.axis_index("s") * mesh.num_cores + lax.axis_index("c")
        lane = lax.iota(jnp.int32, 16)
        neg = jnp.full((16,), -jnp.inf, jnp.float32)

        def init_body(i, _):
            for c in range(nfc):
                acc[pl.ds(i * F + c * 16, 16)] = neg
            return 0

        lax.fori_loop(0, _ACC_ROWS, init_body, 0)

        def chunk_body(ch, _):
            pltpu.sync_copy(srcl_hbm.at[wid, ch], ms)
            pltpu.sync_copy(dstl_hbm.at[wid, ch], md)
            pltpu.sync_copy(cnt_hbm.at[wid, ch], cv)
            cnt = jnp.max(cv[pl.ds(0, 16)])
            ng = jnp.maximum((cnt + 31) // 32, 1)

            def grp_body(g, _):
                cp = pltpu.async_copy(
                    feat_hbm.at[ms.at[pl.ds(g * 32, 32)]], rows, sem)
                d0 = md[pl.ds(g * 32, 16)]
                d1 = md[pl.ds(g * 32 + 16, 16)]
                cp.wait()
                for sub, d in ((0, d0), (1, d1)):
                    for e in range(16):
                        deb = d.at[jnp.full((16,), e, jnp.int32)].get(
                            mode="promise_in_bounds")
                        base = deb * F + lane
                        r = sub * 16 + e
                        for c in range(nfc):
                            idx = base + c * 16
                            av = plsc.load_gather(acc, [idx])
                            mv = jnp.maximum(av, rows[r, pl.ds(c * 16, 16)])
                            plsc.store_scatter(acc, [idx], mv)
                return 0

            lax.fori_loop(0, ng, grp_body, 0)
            return 0

        lax.fori_loop(0, _NCH, chunk_body, 0)
        pltpu.sync_copy(acc.at[pl.ds(0, _NT * F)],
                        out_hbm.at[pl.ds(wid * _NT * F, _NT * F)])

    return seg_kernel(feat, srcl, dstl, cnts)


def _prenet(cfg, node, w0c, w0n, b0, w1, b1):
    def body(cfg_ref, node_ref, w0c_ref, w0n_ref, b0_ref, w1_ref, b1_ref, o_ref):
        h = (jnp.dot(cfg_ref[...], w0c_ref[...], preferred_element_type=jnp.float32)
             + jnp.dot(node_ref[...], w0n_ref[...], preferred_element_type=jnp.float32)
             + b0_ref[...])
        h = _lrelu(h)
        h = jnp.dot(h, w1_ref[...], preferred_element_type=jnp.float32) + b1_ref[...]
        o_ref[...] = _lrelu(h)

    grid = (_N // _R,)
    row = lambda i: (i, 0)
    fixed = lambda i: (0, 0)
    return pl.pallas_call(
        body,
        grid=grid,
        in_specs=[
            pl.BlockSpec((_R, 64), row),
            pl.BlockSpec((_R, 192), row),
            pl.BlockSpec((64, 192), fixed),
            pl.BlockSpec((192, 192), fixed),
            pl.BlockSpec((1, 192), fixed),
            pl.BlockSpec((192, 192), fixed),
            pl.BlockSpec((1, 192), fixed),
        ],
        out_specs=pl.BlockSpec((_R, 192), row),
        out_shape=jax.ShapeDtypeStruct((_N, 192), jnp.float32),
    )(cfg, node, w0c, w0n, b0, w1, b1)


def _combine(aggc, aggx, cfg, x, wlc, wlx, bl, wrc, wrx, wma, bma, wmb, bmb):
    def body(aggc_ref, aggx_ref, cfg_ref, x_ref, wlc_ref, wlx_ref, bl_ref,
             wrc_ref, wrx_ref, wma_ref, bma_ref, wmb_ref, bmb_ref, o_ref):
        ac = aggc_ref[...]
        ax = aggx_ref[...]
        ac = jnp.where(jnp.isneginf(ac), 0.0, ac)
        ax = jnp.where(jnp.isneginf(ax), 0.0, ax)
        xb = x_ref[...]
        h = (jnp.dot(ac, wlc_ref[...], preferred_element_type=jnp.float32)
             + jnp.dot(ax, wlx_ref[...], preferred_element_type=jnp.float32)
             + jnp.dot(cfg_ref[...], wrc_ref[...], preferred_element_type=jnp.float32)
             + jnp.dot(xb, wrx_ref[...], preferred_element_type=jnp.float32)
             + bl_ref[...])
        t = _lrelu(jnp.dot(h, wma_ref[...], preferred_element_type=jnp.float32)
                   + bma_ref[...])
        t = jnp.dot(t, wmb_ref[...], preferred_element_type=jnp.float32) + bmb_ref[...]
        o_ref[...] = xb + _lrelu(t)

    grid = (_N // _R,)
    row = lambda i: (i, 0)
    fixed = lambda i: (0, 0)
    return pl.pallas_call(
        body,
        grid=grid,
        in_specs=[
            pl.BlockSpec((_R, 64), row),
            pl.BlockSpec((_R, 192), row),
            pl.BlockSpec((_R, 64), row),
            pl.BlockSpec((_R, 192), row),
            pl.BlockSpec((64, 256), fixed),
            pl.BlockSpec((192, 256), fixed),
            pl.BlockSpec((1, 256), fixed),
            pl.BlockSpec((64, 256), fixed),
            pl.BlockSpec((192, 256), fixed),
            pl.BlockSpec((256, 192), fixed),
            pl.BlockSpec((1, 192), fixed),
            pl.BlockSpec((192, 192), fixed),
            pl.BlockSpec((1, 192), fixed),
        ],
        out_specs=pl.BlockSpec((_R, 192), row),
        out_shape=jax.ShapeDtypeStruct((_N, 192), jnp.float32),
    )(aggc, aggx, cfg, x, wlc, wlx, bl, wrc, wrx, wma, bma, wmb, bmb)


def kernel(node_features, config_features, edges, Wp0, bp0, Wp1, bp1,
           Wl0, bl0, Wr0, Wma0, bma0, Wmb0, bmb0,
           Wl1, bl1, Wr1, Wma1, bma1, Wmb1, bmb1):
    cfg = config_features
    src = edges[0]
    dst = edges[1]

    srcl, dstl, cnts = _partition(src, dst)
    aggc = _segmax(cfg, srcl, dstl, cnts, 64).reshape(_NPAD, 64)[:_N]
    x = _prenet(cfg, node_features, Wp0[:64], Wp0[64:],
                bp0.reshape(1, -1), Wp1, bp1.reshape(1, -1))

    for (Wl, bl, Wr, Wma, bma, Wmb, bmb) in (
            (Wl0, bl0, Wr0, Wma0, bma0, Wmb0, bmb0),
            (Wl1, bl1, Wr1, Wma1, bma1, Wmb1, bmb1)):
        aggx = _segmax(x, srcl, dstl, cnts, 192).reshape(_NPAD, 192)[:_N]
        x = _combine(aggc, aggx, cfg, x,
                     Wl[:64], Wl[64:], bl.reshape(1, -1),
                     Wr[:64], Wr[64:],
                     Wma, bma.reshape(1, -1), Wmb, bmb.reshape(1, -1))
    return x


# speedup vs baseline: 1.3284x; 1.3284x over previous
import functools

import jax
import jax.numpy as jnp
from jax import lax
from jax.experimental import pallas as pl
from jax.experimental.pallas import tpu as pltpu
from jax.experimental.pallas import tpu_sc as plsc

_N = 10000
_E = 160000
_NW = 32
_NT = 320
_NPAD = _NW * _NT
_C = 8000
_NCH = _E // _C
_CP = _C + 64
_TRASH = _NT
_ACC_ROWS = _NT + 8
_R = 1000


def _lrelu(v):
    return jnp.where(v > 0, v, 0.2 * v)


def _partition(src, dst):
    mesh = plsc.VectorSubcoreMesh(core_axis_name="c", subcore_axis_name="s")

    @functools.partial(
        pl.kernel,
        mesh=mesh,
        out_type=(
            jax.ShapeDtypeStruct((_NW, _NCH, _CP), jnp.int32),
            jax.ShapeDtypeStruct((_NW, _NCH, _CP), jnp.int32),
            jax.ShapeDtypeStruct((_NW, _NCH, 16), jnp.int32),
        ),
        scratch_types=[
            pltpu.VMEM((_C,), jnp.int32),
            pltpu.VMEM((_C,), jnp.int32),
            pltpu.VMEM((_CP,), jnp.int32),
            pltpu.VMEM((_CP,), jnp.int32),
            pltpu.VMEM((16,), jnp.int32),
        ],
        compiler_params=pltpu.CompilerParams(
            needs_layout_passes=False, use_tc_tiling_on_sc=False),
    )
    def part_kernel(src_hbm, dst_hbm, osrc, odst, ocnt, sv, dv, ms, md, cv):
        wid = lax.axis_index("s") * mesh.num_cores + lax.axis_index("c")
        lo = wid * _NT
        hi = lo + _NT
        lane = lax.iota(jnp.int32, 16)

        def ms_init(i, _):
            ms[pl.ds(i * 16, 16)] = jnp.zeros((16,), jnp.int32)
            return 0

        lax.fori_loop(0, _CP // 16, ms_init, 0)

        def chunk_body(ch, _):
            pltpu.sync_copy(src_hbm.at[pl.ds(ch * _C, _C)], sv)
            pltpu.sync_copy(dst_hbm.at[pl.ds(ch * _C, _C)], dv)

            def grp_body(g, cnt):
                d = dv[pl.ds(g * 16, 16)]
                s = sv[pl.ds(g * 16, 16)]
                m = (d >= lo) & (d < hi)
                mi = m.astype(jnp.int32)
                pos = plsc.cumsum(mi) - mi + cnt
                plsc.store_scatter(ms, [pos], s, mask=m)
                plsc.store_scatter(md, [pos], d - lo, mask=m)
                return cnt + jnp.sum(mi)

            cnt = lax.fori_loop(0, _C // 16, grp_body, jnp.int32(0))
            for tg in range(4):
                tpos = cnt + tg * 16 + lane
                plsc.store_scatter(ms, [tpos], jnp.zeros((16,), jnp.int32))
                plsc.store_scatter(md, [tpos], jnp.full((16,), _TRASH, jnp.int32))
            cv[pl.ds(0, 16)] = jnp.full((16,), 1, jnp.int32) * cnt
            pltpu.sync_copy(ms, osrc.at[wid, ch])
            pltpu.sync_copy(md, odst.at[wid, ch])
            pltpu.sync_copy(cv, ocnt.at[wid, ch])
            return 0

        lax.fori_loop(0, _NCH, chunk_body, 0)

    return part_kernel(src, dst)


def _segmax(feat, srcl, dstl, cnts, F):
    mesh = plsc.VectorSubcoreMesh(core_axis_name="c", subcore_axis_name="s")
    nfc = F // 16

    @functools.partial(
        pl.kernel,
        mesh=mesh,
        out_type=jax.ShapeDtypeStruct((_NPAD, F), jnp.float32),
        scratch_types=[
            pltpu.VMEM((_ACC_ROWS, F), jnp.float32),
            pltpu.VMEM((_CP,), jnp.int32),
            pltpu.VMEM((_CP,), jnp.int32),
            pltpu.VMEM((16,), jnp.int32),
            pltpu.VMEM((32, F), jnp.float32),
            pltpu.SemaphoreType.DMA,
        ],
        compiler_params=pltpu.CompilerParams(
            needs_layout_passes=False, use_tc_tiling_on_sc=False),
    )
    def seg_kernel(feat_hbm, srcl_hbm, dstl_hbm, cnt_hbm, out_hbm,
                   acc, ms, md, cv, rows, sem):
        wid = lax.axis_index("s") * mesh.num_cores + lax.axis_index("c")
        lane = lax.iota(jnp.int32, 16)
        neg = jnp.full((16,), -jnp.inf, jnp.float32)

        def init_body(i, _):
            for c in range(nfc):
                acc[i, pl.ds(c * 16, 16)] = neg
            return 0

        lax.fori_loop(0, _ACC_ROWS, init_body, 0)

        def chunk_body(ch, _):
            pltpu.sync_copy(srcl_hbm.at[wid, ch], ms)
            pltpu.sync_copy(dstl_hbm.at[wid, ch], md)
            pltpu.sync_copy(cnt_hbm.at[wid, ch], cv)
            cnt = jnp.max(cv[pl.ds(0, 16)])
            ng = jnp.maximum((cnt + 31) // 32, 1)

            def grp_body(g, _):
                cp = pltpu.async_copy(
                    feat_hbm.at[ms.at[pl.ds(g * 32, 32)]], rows, sem)
                cp.wait()
                for sub in range(2):
                    d = md[pl.ds(g * 32 + sub * 16, 16)]
                    for e in range(16):
                        de = d[e]
                        r = sub * 16 + e
                        for c in range(nfc):
                            sl = pl.ds(c * 16, 16)
                            acc[de, sl] = jnp.maximum(acc[de, sl], rows[r, sl])
                return 0

            lax.fori_loop(0, ng, grp_body, 0)
            return 0

        lax.fori_loop(0, _NCH, chunk_body, 0)
        pltpu.sync_copy(acc.at[pl.ds(0, _NT)], out_hbm.at[pl.ds(wid * _NT, _NT)])

    return seg_kernel(feat, srcl, dstl, cnts)


def _prenet(cfg, node, w0c, w0n, b0, w1, b1):
    def body(cfg_ref, node_ref, w0c_ref, w0n_ref, b0_ref, w1_ref, b1_ref, o_ref):
        h = (jnp.dot(cfg_ref[...], w0c_ref[...], preferred_element_type=jnp.float32)
             + jnp.dot(node_ref[...], w0n_ref[...], preferred_element_type=jnp.float32)
             + b0_ref[...])
        h = _lrelu(h)
        h = jnp.dot(h, w1_ref[...], preferred_element_type=jnp.float32) + b1_ref[...]
        o_ref[...] = _lrelu(h)

    grid = (_N // _R,)
    row = lambda i: (i, 0)
    fixed = lambda i: (0, 0)
    return pl.pallas_call(
        body,
        grid=grid,
        in_specs=[
            pl.BlockSpec((_R, 64), row),
            pl.BlockSpec((_R, 192), row),
            pl.BlockSpec((64, 192), fixed),
            pl.BlockSpec((192, 192), fixed),
            pl.BlockSpec((1, 192), fixed),
            pl.BlockSpec((192, 192), fixed),
            pl.BlockSpec((1, 192), fixed),
        ],
        out_specs=pl.BlockSpec((_R, 192), row),
        out_shape=jax.ShapeDtypeStruct((_N, 192), jnp.float32),
    )(cfg, node, w0c, w0n, b0, w1, b1)


def _combine(aggc, aggx, cfg, x, wlc, wlx, bl, wrc, wrx, wma, bma, wmb, bmb):
    def body(aggc_ref, aggx_ref, cfg_ref, x_ref, wlc_ref, wlx_ref, bl_ref,
             wrc_ref, wrx_ref, wma_ref, bma_ref, wmb_ref, bmb_ref, o_ref):
        ac = aggc_ref[...]
        ax = aggx_ref[...]
        ac = jnp.where(jnp.isneginf(ac), 0.0, ac)
        ax = jnp.where(jnp.isneginf(ax), 0.0, ax)
        xb = x_ref[...]
        h = (jnp.dot(ac, wlc_ref[...], preferred_element_type=jnp.float32)
             + jnp.dot(ax, wlx_ref[...], preferred_element_type=jnp.float32)
             + jnp.dot(cfg_ref[...], wrc_ref[...], preferred_element_type=jnp.float32)
             + jnp.dot(xb, wrx_ref[...], preferred_element_type=jnp.float32)
             + bl_ref[...])
        t = _lrelu(jnp.dot(h, wma_ref[...], preferred_element_type=jnp.float32)
                   + bma_ref[...])
        t = jnp.dot(t, wmb_ref[...], preferred_element_type=jnp.float32) + bmb_ref[...]
        o_ref[...] = xb + _lrelu(t)

    grid = (_N // _R,)
    row = lambda i: (i, 0)
    fixed = lambda i: (0, 0)
    return pl.pallas_call(
        body,
        grid=grid,
        in_specs=[
            pl.BlockSpec((_R, 64), row),
            pl.BlockSpec((_R, 192), row),
            pl.BlockSpec((_R, 64), row),
            pl.BlockSpec((_R, 192), row),
            pl.BlockSpec((64, 256), fixed),
            pl.BlockSpec((192, 256), fixed),
            pl.BlockSpec((1, 256), fixed),
            pl.BlockSpec((64, 256), fixed),
            pl.BlockSpec((192, 256), fixed),
            pl.BlockSpec((256, 192), fixed),
            pl.BlockSpec((1, 192), fixed),
            pl.BlockSpec((192, 192), fixed),
            pl.BlockSpec((1, 192), fixed),
        ],
        out_specs=pl.BlockSpec((_R, 192), row),
        out_shape=jax.ShapeDtypeStruct((_N, 192), jnp.float32),
    )(aggc, aggx, cfg, x, wlc, wlx, bl, wrc, wrx, wma, bma, wmb, bmb)


def kernel(node_features, config_features, edges, Wp0, bp0, Wp1, bp1,
           Wl0, bl0, Wr0, Wma0, bma0, Wmb0, bmb0,
           Wl1, bl1, Wr1, Wma1, bma1, Wmb1, bmb1):
    cfg = config_features
    src = edges[0]
    dst = edges[1]

    srcl, dstl, cnts = _partition(src, dst)
    aggc = _segmax(cfg, srcl, dstl, cnts, 64)[:_N]
    x = _prenet(cfg, node_features, Wp0[:64], Wp0[64:],
                bp0.reshape(1, -1), Wp1, bp1.reshape(1, -1))

    for (Wl, bl, Wr, Wma, bma, Wmb, bmb) in (
            (Wl0, bl0, Wr0, Wma0, bma0, Wmb0, bmb0),
            (Wl1, bl1, Wr1, Wma1, bma1, Wmb1, bmb1)):
        aggx = _segmax(x, srcl, dstl, cnts, 192)[:_N]
        x = _combine(aggc, aggx, cfg, x,
                     Wl[:64], Wl[64:], bl.reshape(1, -1),
                     Wr[:64], Wr[64:],
                     Wma, bma.reshape(1, -1), Wmb, bmb.reshape(1, -1))
    return x


# speedup vs baseline: 1.3634x; 1.0263x over previous
import functools

import jax
import jax.numpy as jnp
from jax import lax
from jax.experimental import pallas as pl
from jax.experimental.pallas import tpu as pltpu
from jax.experimental.pallas import tpu_sc as plsc

_N = 10000
_E = 160000
_NW = 32
_NT = 320
_NPAD = _NW * _NT
_C = 8000
_NCH = _E // _C
_CP = _C + 64
_TRASH = _NT
_ACC_ROWS = _NT + 8
_R = 1000


def _lrelu(v):
    return jnp.where(v > 0, v, 0.2 * v)


def _partition(src, dst):
    mesh = plsc.VectorSubcoreMesh(core_axis_name="c", subcore_axis_name="s")

    @functools.partial(
        pl.kernel,
        mesh=mesh,
        out_type=(
            jax.ShapeDtypeStruct((_NW, _NCH, _CP), jnp.int32),
            jax.ShapeDtypeStruct((_NW, _NCH, _CP), jnp.int32),
            jax.ShapeDtypeStruct((_NW, _NCH, 16), jnp.int32),
        ),
        scratch_types=[
            pltpu.VMEM((_C,), jnp.int32),
            pltpu.VMEM((_C,), jnp.int32),
            pltpu.VMEM((_CP,), jnp.int32),
            pltpu.VMEM((_CP,), jnp.int32),
            pltpu.VMEM((16,), jnp.int32),
        ],
        compiler_params=pltpu.CompilerParams(
            needs_layout_passes=False, use_tc_tiling_on_sc=False),
    )
    def part_kernel(src_hbm, dst_hbm, osrc, odst, ocnt, sv, dv, ms, md, cv):
        wid = lax.axis_index("s") * mesh.num_cores + lax.axis_index("c")
        lo = wid * _NT
        hi = lo + _NT
        lane = lax.iota(jnp.int32, 16)

        def ms_init(i, _):
            ms[pl.ds(i * 16, 16)] = jnp.zeros((16,), jnp.int32)
            return 0

        lax.fori_loop(0, _CP // 16, ms_init, 0)

        def chunk_body(ch, _):
            pltpu.sync_copy(src_hbm.at[pl.ds(ch * _C, _C)], sv)
            pltpu.sync_copy(dst_hbm.at[pl.ds(ch * _C, _C)], dv)

            def grp_body(g, cnt):
                d = dv[pl.ds(g * 16, 16)]
                s = sv[pl.ds(g * 16, 16)]
                m = (d >= lo) & (d < hi)
                mi = m.astype(jnp.int32)
                pos = plsc.cumsum(mi) - mi + cnt
                plsc.store_scatter(ms, [pos], s, mask=m)
                plsc.store_scatter(md, [pos], d - lo, mask=m)
                return cnt + jnp.sum(mi)

            cnt = lax.fori_loop(0, _C // 16, grp_body, jnp.int32(0))
            for tg in range(4):
                tpos = cnt + tg * 16 + lane
                plsc.store_scatter(ms, [tpos], jnp.zeros((16,), jnp.int32))
                plsc.store_scatter(md, [tpos], jnp.full((16,), _TRASH, jnp.int32))
            cv[pl.ds(0, 16)] = jnp.full((16,), 1, jnp.int32) * cnt
            pltpu.sync_copy(ms, osrc.at[wid, ch])
            pltpu.sync_copy(md, odst.at[wid, ch])
            pltpu.sync_copy(cv, ocnt.at[wid, ch])
            return 0

        lax.fori_loop(0, _NCH, chunk_body, 0)

    return part_kernel(src, dst)


def _segmax(feat, srcl, dstl, cnts, F):
    mesh = plsc.VectorSubcoreMesh(core_axis_name="c", subcore_axis_name="s")
    nfc = F // 16

    @functools.partial(
        pl.kernel,
        mesh=mesh,
        out_type=jax.ShapeDtypeStruct((_NPAD, F), jnp.float32),
        scratch_types=[
            pltpu.VMEM((_ACC_ROWS, F), jnp.float32),
            pltpu.VMEM((_CP,), jnp.int32),
            pltpu.VMEM((_CP,), jnp.int32),
            pltpu.VMEM((16,), jnp.int32),
            pltpu.VMEM((32, F), jnp.float32),
            pltpu.SemaphoreType.DMA,
        ],
        compiler_params=pltpu.CompilerParams(
            needs_layout_passes=False, use_tc_tiling_on_sc=False),
    )
    def seg_kernel(feat_hbm, srcl_hbm, dstl_hbm, cnt_hbm, out_hbm,
                   acc, ms, md, cv, rows, sem):
        wid = lax.axis_index("s") * mesh.num_cores + lax.axis_index("c")
        lane = lax.iota(jnp.int32, 16)
        neg = jnp.full((16,), -jnp.inf, jnp.float32)

        def init_body(i, _):
            for c in range(nfc):
                acc[i, pl.ds(c * 16, 16)] = neg
            return 0

        lax.fori_loop(0, _ACC_ROWS, init_body, 0)

        def chunk_body(ch, _):
            pltpu.sync_copy(srcl_hbm.at[wid, ch], ms)
            pltpu.sync_copy(dstl_hbm.at[wid, ch], md)
            pltpu.sync_copy(cnt_hbm.at[wid, ch], cv)
            cnt = jnp.max(cv[pl.ds(0, 16)])
            ng = jnp.maximum((cnt + 31) // 32, 1)

            def grp_body(g, _):
                cp = pltpu.async_copy(
                    feat_hbm.at[ms.at[pl.ds(g * 32, 32)]], rows, sem)
                cp.wait()
                for sub in range(2):
                    d = md[pl.ds(g * 32 + sub * 16, 16)]
                    for e in range(16):
                        de = d[e]
                        r = sub * 16 + e
                        for c in range(nfc):
                            sl = pl.ds(c * 16, 16)
                            acc[de, sl] = jnp.maximum(acc[de, sl], rows[r, sl])
                return 0

            lax.fori_loop(0, ng, grp_body, 0)
            return 0

        lax.fori_loop(0, _NCH, chunk_body, 0)
        pltpu.sync_copy(acc.at[pl.ds(0, _NT)], out_hbm.at[pl.ds(wid * _NT, _NT)])

    return seg_kernel(feat, srcl, dstl, cnts)


def _prenet(cfg, node, w0c, w0n, b0, w1, b1):
    def body(cfg_ref, node_ref, w0c_ref, w0n_ref, b0_ref, w1_ref, b1_ref, o_ref):
        h = (jnp.dot(cfg_ref[...], w0c_ref[...], preferred_element_type=jnp.float32)
             + jnp.dot(node_ref[...], w0n_ref[...], preferred_element_type=jnp.float32)
             + b0_ref[...])
        h = _lrelu(h)
        h = jnp.dot(h, w1_ref[...], preferred_element_type=jnp.float32) + b1_ref[...]
        o_ref[...] = _lrelu(h)

    grid = (_N // _R,)
    row = lambda i: (i, 0)
    fixed = lambda i: (0, 0)
    return pl.pallas_call(
        body,
        grid=grid,
        in_specs=[
            pl.BlockSpec((_R, 64), row),
            pl.BlockSpec((_R, 192), row),
            pl.BlockSpec((64, 192), fixed),
            pl.BlockSpec((192, 192), fixed),
            pl.BlockSpec((1, 192), fixed),
            pl.BlockSpec((192, 192), fixed),
            pl.BlockSpec((1, 192), fixed),
        ],
        out_specs=pl.BlockSpec((_R, 192), row),
        out_shape=jax.ShapeDtypeStruct((_N, 192), jnp.float32),
    )(cfg, node, w0c, w0n, b0, w1, b1)


def _combine(aggc, aggx, cfg, x, wlc, wlx, bl, wrc, wrx, wma, bma, wmb, bmb):
    def body(aggc_ref, aggx_ref, cfg_ref, x_ref, wlc_ref, wlx_ref, bl_ref,
             wrc_ref, wrx_ref, wma_ref, bma_ref, wmb_ref, bmb_ref, o_ref):
        ac = aggc_ref[...]
        ax = aggx_ref[...]
        ac = jnp.where(jnp.isneginf(ac), 0.0, ac)
        ax = jnp.where(jnp.isneginf(ax), 0.0, ax)
        xb = x_ref[...]
        h = (jnp.dot(ac, wlc_ref[...], preferred_element_type=jnp.float32)
             + jnp.dot(ax, wlx_ref[...], preferred_element_type=jnp.float32)
             + jnp.dot(cfg_ref[...], wrc_ref[...], preferred_element_type=jnp.float32)
             + jnp.dot(xb, wrx_ref[...], preferred_element_type=jnp.float32)
             + bl_ref[...])
        t = _lrelu(jnp.dot(h, wma_ref[...], preferred_element_type=jnp.float32)
                   + bma_ref[...])
        t = jnp.dot(t, wmb_ref[...], preferred_element_type=jnp.float32) + bmb_ref[...]
        o_ref[...] = xb + _lrelu(t)

    grid = (_N // _R,)
    row = lambda i: (i, 0)
    fixed = lambda i: (0, 0)
    return pl.pallas_call(
        body,
        grid=grid,
        in_specs=[
            pl.BlockSpec((_R, 64), row),
            pl.BlockSpec((_R, 192), row),
            pl.BlockSpec((_R, 64), row),
            pl.BlockSpec((_R, 192), row),
            pl.BlockSpec((64, 256), fixed),
            pl.BlockSpec((192, 256), fixed),
            pl.BlockSpec((1, 256), fixed),
            pl.BlockSpec((64, 256), fixed),
            pl.BlockSpec((192, 256), fixed),
            pl.BlockSpec((256, 192), fixed),
            pl.BlockSpec((1, 192), fixed),
            pl.BlockSpec((192, 192), fixed),
            pl.BlockSpec((1, 192), fixed),
        ],
        out_specs=pl.BlockSpec((_R, 192), row),
        out_shape=jax.ShapeDtypeStruct((_N, 192), jnp.float32),
    )(aggc, aggx, cfg, x, wlc, wlx, bl, wrc, wrx, wma, bma, wmb, bmb)


def kernel(node_features, config_features, edges, Wp0, bp0, Wp1, bp1,
           Wl0, bl0, Wr0, Wma0, bma0, Wmb0, bmb0,
           Wl1, bl1, Wr1, Wma1, bma1, Wmb1, bmb1):
    cfg = config_features
    src = edges[0]
    dst = edges[1]

    srcl, dstl, cnts = _partition(src, dst)
    x = _prenet(cfg, node_features, Wp0[:64], Wp0[64:],
                bp0.reshape(1, -1), Wp1, bp1.reshape(1, -1))

    agg1 = _segmax(jnp.concatenate([cfg, x], axis=1), srcl, dstl, cnts, 256)
    aggc = agg1[:_N, :64]
    x = _combine(aggc, agg1[:_N, 64:], cfg, x,
                 Wl0[:64], Wl0[64:], bl0.reshape(1, -1),
                 Wr0[:64], Wr0[64:],
                 Wma0, bma0.reshape(1, -1), Wmb0, bmb0.reshape(1, -1))

    aggx2 = _segmax(x, srcl, dstl, cnts, 192)[:_N]
    x = _combine(aggc, aggx2, cfg, x,
                 Wl1[:64], Wl1[64:], bl1.reshape(1, -1),
                 Wr1[:64], Wr1[64:],
                 Wma1, bma1.reshape(1, -1), Wmb1, bmb1.reshape(1, -1))
    return x


# speedup vs baseline: 1.5187x; 1.1139x over previous
import functools

import jax
import jax.numpy as jnp
from jax import lax
from jax.experimental import pallas as pl
from jax.experimental.pallas import tpu as pltpu
from jax.experimental.pallas import tpu_sc as plsc

_N = 10000
_E = 160000
_NW = 32
_NT = 320
_NPAD = _NW * _NT
_C = 16000
_NCH = _E // _C
_CP = _C + 64
_TRASH = _NT
_ACC_ROWS = _NT + 8
_R = 1000


def _lrelu(v):
    return jnp.where(v > 0, v, 0.2 * v)


def _partition(src, dst):
    mesh = plsc.VectorSubcoreMesh(core_axis_name="c", subcore_axis_name="s")

    @functools.partial(
        pl.kernel,
        mesh=mesh,
        out_type=(
            jax.ShapeDtypeStruct((_NW, _NCH, _CP), jnp.int32),
            jax.ShapeDtypeStruct((_NW, _NCH, _CP), jnp.int32),
            jax.ShapeDtypeStruct((_NW, _NCH, 16), jnp.int32),
        ),
        scratch_types=[
            pltpu.VMEM((_C,), jnp.int32),
            pltpu.VMEM((_C,), jnp.int32),
            pltpu.VMEM((_CP,), jnp.int32),
            pltpu.VMEM((_CP,), jnp.int32),
            pltpu.VMEM((16,), jnp.int32),
        ],
        compiler_params=pltpu.CompilerParams(
            needs_layout_passes=False, use_tc_tiling_on_sc=False),
    )
    def part_kernel(src_hbm, dst_hbm, osrc, odst, ocnt, sv, dv, ms, md, cv):
        wid = lax.axis_index("s") * mesh.num_cores + lax.axis_index("c")
        lo = wid * _NT
        hi = lo + _NT
        lane = lax.iota(jnp.int32, 16)

        def ms_init(i, _):
            ms[pl.ds(i * 16, 16)] = jnp.zeros((16,), jnp.int32)
            return 0

        lax.fori_loop(0, _CP // 16, ms_init, 0)

        def chunk_body(ch, _):
            pltpu.sync_copy(src_hbm.at[pl.ds(ch * _C, _C)], sv)
            pltpu.sync_copy(dst_hbm.at[pl.ds(ch * _C, _C)], dv)

            def grp_body(g, cnt):
                d = dv[pl.ds(g * 16, 16)]
                s = sv[pl.ds(g * 16, 16)]
                m = (d >= lo) & (d < hi)
                mi = m.astype(jnp.int32)
                pos = plsc.cumsum(mi) - mi + cnt
                plsc.store_scatter(ms, [pos], s, mask=m)
                plsc.store_scatter(md, [pos], d - lo, mask=m)
                return cnt + jnp.sum(mi)

            cnt = lax.fori_loop(0, _C // 16, grp_body, jnp.int32(0))
            for tg in range(4):
                tpos = cnt + tg * 16 + lane
                plsc.store_scatter(ms, [tpos], jnp.zeros((16,), jnp.int32))
                plsc.store_scatter(md, [tpos], jnp.full((16,), _TRASH, jnp.int32))
            cv[pl.ds(0, 16)] = jnp.full((16,), 1, jnp.int32) * cnt
            pltpu.sync_copy(ms, osrc.at[wid, ch])
            pltpu.sync_copy(md, odst.at[wid, ch])
            pltpu.sync_copy(cv, ocnt.at[wid, ch])
            return 0

        lax.fori_loop(0, _NCH, chunk_body, 0)

    return part_kernel(src, dst)


def _segmax(feat, srcl, dstl, cnts, F):
    mesh = plsc.VectorSubcoreMesh(core_axis_name="c", subcore_axis_name="s")
    nfc = F // 16

    @functools.partial(
        pl.kernel,
        mesh=mesh,
        out_type=jax.ShapeDtypeStruct((_NPAD, F), jnp.float32),
        scratch_types=[
            pltpu.VMEM((_ACC_ROWS, F), jnp.float32),
            pltpu.VMEM((_CP,), jnp.int32),
            pltpu.VMEM((_CP,), jnp.int32),
            pltpu.VMEM((16,), jnp.int32),
            pltpu.VMEM((32, F), jnp.float32),
            pltpu.SemaphoreType.DMA,
        ],
        compiler_params=pltpu.CompilerParams(
            needs_layout_passes=False, use_tc_tiling_on_sc=False),
    )
    def seg_kernel(feat_hbm, srcl_hbm, dstl_hbm, cnt_hbm, out_hbm,
                   acc, ms, md, cv, rows, sem):
        wid = lax.axis_index("s") * mesh.num_cores + lax.axis_index("c")
        lane = lax.iota(jnp.int32, 16)
        neg = jnp.full((16,), -jnp.inf, jnp.float32)

        def init_body(i, _):
            for c in range(nfc):
                acc[i, pl.ds(c * 16, 16)] = neg
            return 0

        lax.fori_loop(0, _ACC_ROWS, init_body, 0)

        def chunk_body(ch, _):
            pltpu.sync_copy(srcl_hbm.at[wid, ch], ms)
            pltpu.sync_copy(dstl_hbm.at[wid, ch], md)
            pltpu.sync_copy(cnt_hbm.at[wid, ch], cv)
            cnt = jnp.max(cv[pl.ds(0, 16)])
            ng = jnp.maximum((cnt + 31) // 32, 1)

            def grp_body(g, _):
                cp = pltpu.async_copy(
                    feat_hbm.at[ms.at[pl.ds(g * 32, 32)]], rows, sem)
                cp.wait()
                for sub in range(2):
                    d = md[pl.ds(g * 32 + sub * 16, 16)]
                    for e in range(16):
                        de = d[e]
                        r = sub * 16 + e
                        for c in range(nfc):
                            sl = pl.ds(c * 16, 16)
                            acc[de, sl] = jnp.maximum(acc[de, sl], rows[r, sl])
                return 0

            lax.fori_loop(0, ng, grp_body, 0)
            return 0

        lax.fori_loop(0, _NCH, chunk_body, 0)
        pltpu.sync_copy(acc.at[pl.ds(0, _NT)], out_hbm.at[pl.ds(wid * _NT, _NT)])

    return seg_kernel(feat, srcl, dstl, cnts)


def _prenet(cfg, node, w0c, w0n, b0, w1, b1):
    def body(cfg_ref, node_ref, w0c_ref, w0n_ref, b0_ref, w1_ref, b1_ref, o_ref):
        h = (jnp.dot(cfg_ref[...], w0c_ref[...], preferred_element_type=jnp.float32)
             + jnp.dot(node_ref[...], w0n_ref[...], preferred_element_type=jnp.float32)
             + b0_ref[...])
        h = _lrelu(h)
        h = jnp.dot(h, w1_ref[...], preferred_element_type=jnp.float32) + b1_ref[...]
        o_ref[...] = _lrelu(h)

    grid = (_N // _R,)
    row = lambda i: (i, 0)
    fixed = lambda i: (0, 0)
    return pl.pallas_call(
        body,
        grid=grid,
        in_specs=[
            pl.BlockSpec((_R, 64), row),
            pl.BlockSpec((_R, 192), row),
            pl.BlockSpec((64, 192), fixed),
            pl.BlockSpec((192, 192), fixed),
            pl.BlockSpec((1, 192), fixed),
            pl.BlockSpec((192, 192), fixed),
            pl.BlockSpec((1, 192), fixed),
        ],
        out_specs=pl.BlockSpec((_R, 192), row),
        out_shape=jax.ShapeDtypeStruct((_N, 192), jnp.float32),
    )(cfg, node, w0c, w0n, b0, w1, b1)


def _combine(aggc, aggx, cfg, x, wlc, wlx, bl, wrc, wrx, wma, bma, wmb, bmb):
    def body(aggc_ref, aggx_ref, cfg_ref, x_ref, wlc_ref, wlx_ref, bl_ref,
             wrc_ref, wrx_ref, wma_ref, bma_ref, wmb_ref, bmb_ref, o_ref):
        ac = aggc_ref[...]
        ax = aggx_ref[...]
        ac = jnp.where(jnp.isneginf(ac), 0.0, ac)
        ax = jnp.where(jnp.isneginf(ax), 0.0, ax)
        xb = x_ref[...]
        h = (jnp.dot(ac, wlc_ref[...], preferred_element_type=jnp.float32)
             + jnp.dot(ax, wlx_ref[...], preferred_element_type=jnp.float32)
             + jnp.dot(cfg_ref[...], wrc_ref[...], preferred_element_type=jnp.float32)
             + jnp.dot(xb, wrx_ref[...], preferred_element_type=jnp.float32)
             + bl_ref[...])
        t = _lrelu(jnp.dot(h, wma_ref[...], preferred_element_type=jnp.float32)
                   + bma_ref[...])
        t = jnp.dot(t, wmb_ref[...], preferred_element_type=jnp.float32) + bmb_ref[...]
        o_ref[...] = xb + _lrelu(t)

    grid = (_N // _R,)
    row = lambda i: (i, 0)
    fixed = lambda i: (0, 0)
    return pl.pallas_call(
        body,
        grid=grid,
        in_specs=[
            pl.BlockSpec((_R, 64), row),
            pl.BlockSpec((_R, 192), row),
            pl.BlockSpec((_R, 64), row),
            pl.BlockSpec((_R, 192), row),
            pl.BlockSpec((64, 256), fixed),
            pl.BlockSpec((192, 256), fixed),
            pl.BlockSpec((1, 256), fixed),
            pl.BlockSpec((64, 256), fixed),
            pl.BlockSpec((192, 256), fixed),
            pl.BlockSpec((256, 192), fixed),
            pl.BlockSpec((1, 192), fixed),
            pl.BlockSpec((192, 192), fixed),
            pl.BlockSpec((1, 192), fixed),
        ],
        out_specs=pl.BlockSpec((_R, 192), row),
        out_shape=jax.ShapeDtypeStruct((_N, 192), jnp.float32),
    )(aggc, aggx, cfg, x, wlc, wlx, bl, wrc, wrx, wma, bma, wmb, bmb)


def kernel(node_features, config_features, edges, Wp0, bp0, Wp1, bp1,
           Wl0, bl0, Wr0, Wma0, bma0, Wmb0, bmb0,
           Wl1, bl1, Wr1, Wma1, bma1, Wmb1, bmb1):
    cfg = config_features
    src = edges[0]
    dst = edges[1]

    srcl, dstl, cnts = _partition(src, dst)
    x = _prenet(cfg, node_features, Wp0[:64], Wp0[64:],
                bp0.reshape(1, -1), Wp1, bp1.reshape(1, -1))

    agg1 = _segmax(jnp.concatenate([cfg, x], axis=1), srcl, dstl, cnts, 256)
    aggc = agg1[:_N, :64]
    x = _combine(aggc, agg1[:_N, 64:], cfg, x,
                 Wl0[:64], Wl0[64:], bl0.reshape(1, -1),
                 Wr0[:64], Wr0[64:],
                 Wma0, bma0.reshape(1, -1), Wmb0, bmb0.reshape(1, -1))

    aggx2 = _segmax(x, srcl, dstl, cnts, 192)[:_N]
    x = _combine(aggc, aggx2, cfg, x,
                 Wl1[:64], Wl1[64:], bl1.reshape(1, -1),
                 Wr1[:64], Wr1[64:],
                 Wma1, bma1.reshape(1, -1), Wmb1, bmb1.reshape(1, -1))
    return x
